# Initial kernel scaffold; baseline (speedup 1.0000x reference)
import jax
import jax.numpy as jnp
from jax.experimental import pallas as pl


def kernel(inp, feature_idx):
    raise NotImplementedError("write your pallas kernel here")


# speedup vs baseline: 4.2698x; 4.2698x over previous
import jax
import jax.numpy as jnp
from jax.experimental import pallas as pl
from jax.experimental.pallas import tpu as pltpu

N, D, F = 65536, 512, 128
BR = 2048
NB = N // BR


def _minmax_body(x_ref, mn_ref, mx_ref):
    i = pl.program_id(0)
    x = x_ref[...].reshape(BR // 8, 8, D)
    pmn = jnp.min(x, axis=0)
    pmx = jnp.max(x, axis=0)

    @pl.when(i == 0)
    def _():
        mn_ref[...] = pmn
        mx_ref[...] = pmx

    @pl.when(i > 0)
    def _():
        mn_ref[...] = jnp.minimum(mn_ref[...], pmn)
        mx_ref[...] = jnp.maximum(mx_ref[...], pmx)


def _apply_body(idx_ref, mn_ref, mx_ref, x_ref, o_ref):
    ci = jax.lax.broadcasted_iota(jnp.int32, (F, D), 1)
    sel = jnp.any(ci == idx_ref[...], axis=0, keepdims=True)
    mn = jnp.min(mn_ref[...], axis=0, keepdims=True)
    mx = jnp.max(mx_ref[...], axis=0, keepdims=True)
    rs = 1.0 / (mx - mn)
    a = jnp.where(sel, rs, 1.0)
    b = jnp.where(sel, -mn * rs, 0.0)
    o_ref[...] = x_ref[...] * a + b


def kernel(inp, feature_idx):
    idx2d = feature_idx.astype(jnp.int32).reshape(F, 1)

    mn8, mx8 = pl.pallas_call(
        _minmax_body,
        grid=(NB,),
        in_specs=[pl.BlockSpec((BR, D), lambda i: (i, 0))],
        out_specs=[
            pl.BlockSpec((8, D), lambda i: (0, 0)),
            pl.BlockSpec((8, D), lambda i: (0, 0)),
        ],
        out_shape=[
            jax.ShapeDtypeStruct((8, D), jnp.float32),
            jax.ShapeDtypeStruct((8, D), jnp.float32),
        ],
        compiler_params=pltpu.CompilerParams(
            dimension_semantics=("arbitrary",)),
    )(inp)

    out = pl.pallas_call(
        _apply_body,
        grid=(NB,),
        in_specs=[
            pl.BlockSpec((F, 1), lambda i: (0, 0)),
            pl.BlockSpec((8, D), lambda i: (0, 0)),
            pl.BlockSpec((8, D), lambda i: (0, 0)),
            pl.BlockSpec((BR, D), lambda i: (i, 0)),
        ],
        out_specs=pl.BlockSpec((BR, D), lambda i: (i, 0)),
        out_shape=jax.ShapeDtypeStruct((N, D), jnp.float32),
        compiler_params=pltpu.CompilerParams(
            dimension_semantics=("parallel",)),
    )(idx2d, mn8, mx8, inp)
    return out


# speedup vs baseline: 4.4417x; 1.0403x over previous
import jax
import jax.numpy as jnp
from jax.experimental import pallas as pl
from jax.experimental.pallas import tpu as pltpu

N, D, F = 65536, 512, 128
BR = 4096
NB = N // BR


def _minmax_body(x_ref, mn_ref, mx_ref):
    i = pl.program_id(0)
    x = x_ref[...].reshape(BR // 8, 8, D)
    pmn = jnp.min(x, axis=0)
    pmx = jnp.max(x, axis=0)

    @pl.when(i == 0)
    def _():
        mn_ref[...] = pmn
        mx_ref[...] = pmx

    @pl.when(i > 0)
    def _():
        mn_ref[...] = jnp.minimum(mn_ref[...], pmn)
        mx_ref[...] = jnp.maximum(mx_ref[...], pmx)


def _apply_body(idx_ref, mn_ref, mx_ref, x_ref, o_ref):
    ci = jax.lax.broadcasted_iota(jnp.int32, (F, D), 1)
    sel = jnp.any(ci == idx_ref[...], axis=0, keepdims=True)
    mn = jnp.min(mn_ref[...], axis=0, keepdims=True)
    mx = jnp.max(mx_ref[...], axis=0, keepdims=True)
    rs = 1.0 / (mx - mn)
    a = jnp.where(sel, rs, 1.0)
    b = jnp.where(sel, -mn * rs, 0.0)
    o_ref[...] = x_ref[...] * a + b


def kernel(inp, feature_idx):
    idx2d = feature_idx.astype(jnp.int32).reshape(F, 1)

    mn8, mx8 = pl.pallas_call(
        _minmax_body,
        grid=(NB,),
        in_specs=[pl.BlockSpec((BR, D), lambda i: (i, 0))],
        out_specs=[
            pl.BlockSpec((8, D), lambda i: (0, 0)),
            pl.BlockSpec((8, D), lambda i: (0, 0)),
        ],
        out_shape=[
            jax.ShapeDtypeStruct((8, D), jnp.float32),
            jax.ShapeDtypeStruct((8, D), jnp.float32),
        ],
        compiler_params=pltpu.CompilerParams(
            dimension_semantics=("arbitrary",)),
    )(inp)

    out = pl.pallas_call(
        _apply_body,
        grid=(NB,),
        in_specs=[
            pl.BlockSpec((F, 1), lambda i: (0, 0)),
            pl.BlockSpec((8, D), lambda i: (0, 0)),
            pl.BlockSpec((8, D), lambda i: (0, 0)),
            pl.BlockSpec((BR, D), lambda i: (i, 0)),
        ],
        out_specs=pl.BlockSpec((BR, D), lambda i: (i, 0)),
        out_shape=jax.ShapeDtypeStruct((N, D), jnp.float32),
        compiler_params=pltpu.CompilerParams(
            dimension_semantics=("parallel",)),
    )(idx2d, mn8, mx8, inp)
    return out
